# baseline (speedup 1.0000x reference)
import jax
import jax.numpy as jnp
from jax import lax
from jax.experimental import pallas as pl
from jax.experimental.pallas import tpu as pltpu
from jax.experimental.pallas import tpu_sc as plsc

B, S, H = 4, 2048, 1024
MAX_DEPTH = 16
PU = 64
TAG_V = 256
SUB_V = 1024
EPS = 1e-12

N = B * S
R = N * MAX_DEPTH

NC, NS = 2, 16
NW = NC * NS
CHUNK = 128


def _sc_gather(tag_tab_flat, sub_tab_flat, tag_idx_flat, sub_idx_flat):
    rows = tag_idx_flat.shape[0]
    dt = tag_tab_flat.dtype
    per_w = rows // NW
    n_chunks = per_w // CHUNK

    def body(tag_tab, sub_tab, tag_idx, sub_idx, tag_out, sub_out,
             idx_t, idx_s, rows_t, rows_s, sem_t, sem_s):
        wid = lax.axis_index("s") * NC + lax.axis_index("c")
        base = wid * per_w

        def chunk(k, carry):
            off = base + k * CHUNK
            pltpu.sync_copy(tag_idx.at[pl.ds(off, CHUNK)], idx_t)
            pltpu.sync_copy(sub_idx.at[pl.ds(off, CHUNK)], idx_s)
            ct = pltpu.async_copy(tag_tab.at[idx_t], rows_t, sem_t)
            cs = pltpu.async_copy(sub_tab.at[idx_s], rows_s, sem_s)
            ct.wait()
            pltpu.sync_copy(rows_t, tag_out.at[pl.ds(off, CHUNK)])
            cs.wait()
            pltpu.sync_copy(rows_s, sub_out.at[pl.ds(off, CHUNK)])
            return carry

        lax.fori_loop(0, n_chunks, chunk, 0)

    mesh = plsc.VectorSubcoreMesh(core_axis_name="c", subcore_axis_name="s")
    return pl.kernel(
        body,
        out_type=(jax.ShapeDtypeStruct((rows, PU), dt),
                  jax.ShapeDtypeStruct((rows, PU), dt)),
        mesh=mesh,
        scratch_types=[
            pltpu.VMEM((CHUNK,), jnp.int32),
            pltpu.VMEM((CHUNK,), jnp.int32),
            pltpu.VMEM((CHUNK, PU), dt),
            pltpu.VMEM((CHUNK, PU), dt),
            pltpu.SemaphoreType.DMA,
            pltpu.SemaphoreType.DMA,
        ],
        compiler_params=pltpu.CompilerParams(use_tc_tiling_on_sc=False),
    )(tag_tab_flat, sub_tab_flat, tag_idx_flat, sub_idx_flat)


TB = 512


def _tc_body(tag_ref, sub_ref, x_ref, pos_ref, wi_ref, bi_ref, wo_ref, bo_ref,
             g_ref, b_ref, o_ref):
    path = tag_ref[...] + sub_ref[...]
    h1 = jnp.dot(path, wi_ref[...], preferred_element_type=jnp.float32)
    h1 = jnp.maximum(h1 + bi_ref[...], 0.0).astype(jnp.bfloat16)
    pe = jnp.dot(h1, wo_ref[...], preferred_element_type=jnp.float32)
    emb = x_ref[...] + pos_ref[...] + (pe + bo_ref[...])
    mean = jnp.mean(emb, axis=-1, keepdims=True)
    cent = emb - mean
    var = jnp.mean(cent * cent, axis=-1, keepdims=True)
    o_ref[...] = cent * lax.rsqrt(var + EPS) * g_ref[...] + b_ref[...]


def _tc_fused(tag2d, sub2d, x2d, pos, wi, bi, wo, bo, gamma, beta):
    grid = (S // TB, B)
    dp = MAX_DEPTH * PU
    tok = lambda s, b: (b * (S // TB) + s, 0)
    return pl.pallas_call(
        _tc_body,
        grid=grid,
        in_specs=[
            pl.BlockSpec((TB, dp), tok),
            pl.BlockSpec((TB, dp), tok),
            pl.BlockSpec((TB, H), tok),
            pl.BlockSpec((TB, H), lambda s, b: (s, 0)),
            pl.BlockSpec((dp, 4 * H), lambda s, b: (0, 0)),
            pl.BlockSpec((1, 4 * H), lambda s, b: (0, 0)),
            pl.BlockSpec((4 * H, H), lambda s, b: (0, 0)),
            pl.BlockSpec((1, H), lambda s, b: (0, 0)),
            pl.BlockSpec((1, H), lambda s, b: (0, 0)),
            pl.BlockSpec((1, H), lambda s, b: (0, 0)),
        ],
        out_specs=pl.BlockSpec((TB, H), tok),
        out_shape=jax.ShapeDtypeStruct((N, H), jnp.float32),
        compiler_params=pltpu.CompilerParams(
            dimension_semantics=("arbitrary", "arbitrary")),
    )(tag2d, sub2d, x2d, pos, wi, bi, wo, bo, gamma, beta)


def kernel(inputs_embeds, path_tags_seq, path_subs_seq, tag_tables,
           subs_tables, W_inner, b_inner, W_out, b_out, pos_emb, ln_gamma,
           ln_beta):
    seq_len = inputs_embeds.shape[1]
    depths = jnp.arange(MAX_DEPTH, dtype=jnp.int32)
    tag_idx = (path_tags_seq + depths * TAG_V).reshape(R)
    sub_idx = (path_subs_seq + depths * SUB_V).reshape(R)
    tag_flat = tag_tables.reshape(MAX_DEPTH * TAG_V, PU).astype(jnp.bfloat16)
    sub_flat = subs_tables.reshape(MAX_DEPTH * SUB_V, PU).astype(jnp.bfloat16)

    x2d = inputs_embeds.reshape(N, H)
    pos = pos_emb[:seq_len]
    wi = W_inner.astype(jnp.bfloat16)
    bi = b_inner.reshape(1, 4 * H)
    wo = W_out.astype(jnp.bfloat16)
    bo = b_out.reshape(1, H)
    g = ln_gamma.reshape(1, H)
    bt = ln_beta.reshape(1, H)

    tag_cat, sub_cat = _sc_gather(tag_flat, sub_flat, tag_idx, sub_idx)
    out = _tc_fused(
        tag_cat.reshape(N, MAX_DEPTH * PU),
        sub_cat.reshape(N, MAX_DEPTH * PU),
        x2d, pos, wi, bi, wo, bo, g, bt)
    return out.reshape(B, S, H)


# speedup vs baseline: 1.1752x; 1.1752x over previous
import jax
import jax.numpy as jnp
from jax import lax
from jax.experimental import pallas as pl
from jax.experimental.pallas import tpu as pltpu
from jax.experimental.pallas import tpu_sc as plsc

B, S, H = 4, 2048, 1024
MAX_DEPTH = 16
PU = 64
TAG_V = 256
SUB_V = 1024
EPS = 1e-12

N = B * S
R = N * MAX_DEPTH

NC, NS = 2, 16
NW = NC * NS
CHUNK = 128


def _sc_gather(tag_tab_flat, sub_tab_flat, tag_idx_flat, sub_idx_flat):
    rows = tag_idx_flat.shape[0]
    dt = tag_tab_flat.dtype
    per_w = rows // NW
    n_chunks = per_w // CHUNK

    def body(tag_tab, sub_tab, tag_idx, sub_idx, tag_out, sub_out,
             idx_t, idx_s, rows_t, rows_s, sem_t, sem_s):
        wid = lax.axis_index("s") * NC + lax.axis_index("c")
        base = wid * per_w

        def chunk(k, carry):
            off = base + k * CHUNK
            pltpu.sync_copy(tag_idx.at[pl.ds(off, CHUNK)], idx_t)
            pltpu.sync_copy(sub_idx.at[pl.ds(off, CHUNK)], idx_s)
            ct = pltpu.async_copy(tag_tab.at[idx_t], rows_t, sem_t)
            cs = pltpu.async_copy(sub_tab.at[idx_s], rows_s, sem_s)
            ct.wait()
            pltpu.sync_copy(rows_t, tag_out.at[pl.ds(off, CHUNK)])
            cs.wait()
            pltpu.sync_copy(rows_s, sub_out.at[pl.ds(off, CHUNK)])
            return carry

        lax.fori_loop(0, n_chunks, chunk, 0)

    mesh = plsc.VectorSubcoreMesh(core_axis_name="c", subcore_axis_name="s")
    return pl.kernel(
        body,
        out_type=(jax.ShapeDtypeStruct((rows, PU), dt),
                  jax.ShapeDtypeStruct((rows, PU), dt)),
        mesh=mesh,
        scratch_types=[
            pltpu.VMEM((CHUNK,), jnp.int32),
            pltpu.VMEM((CHUNK,), jnp.int32),
            pltpu.VMEM((CHUNK, PU), dt),
            pltpu.VMEM((CHUNK, PU), dt),
            pltpu.SemaphoreType.DMA,
            pltpu.SemaphoreType.DMA,
        ],
        compiler_params=pltpu.CompilerParams(use_tc_tiling_on_sc=False),
    )(tag_tab_flat, sub_tab_flat, tag_idx_flat, sub_idx_flat)


TB = 512


def _tc_body(tag_ref, sub_ref, x_ref, pos_ref, wi_ref, bi_ref, wo_ref, bo_ref,
             g_ref, b_ref, o_ref):
    path = (tag_ref[...] + sub_ref[...]).astype(jnp.bfloat16)
    h1 = jnp.dot(path, wi_ref[...], preferred_element_type=jnp.float32)
    h1 = jnp.maximum(h1 + bi_ref[...], 0.0).astype(jnp.bfloat16)
    pe = jnp.dot(h1, wo_ref[...], preferred_element_type=jnp.float32)
    emb = x_ref[...] + pos_ref[...] + (pe + bo_ref[...])
    mean = jnp.mean(emb, axis=-1, keepdims=True)
    cent = emb - mean
    var = jnp.mean(cent * cent, axis=-1, keepdims=True)
    o_ref[...] = cent * lax.rsqrt(var + EPS) * g_ref[...] + b_ref[...]


def _tc_fused(tag2d, sub2d, x2d, pos, wi, bi, wo, bo, gamma, beta):
    grid = (S // TB, B)
    dp = MAX_DEPTH * PU
    tok = lambda s, b: (b * (S // TB) + s, 0)
    return pl.pallas_call(
        _tc_body,
        grid=grid,
        in_specs=[
            pl.BlockSpec((TB, dp), tok),
            pl.BlockSpec((TB, dp), tok),
            pl.BlockSpec((TB, H), tok),
            pl.BlockSpec((TB, H), lambda s, b: (s, 0)),
            pl.BlockSpec((dp, 4 * H), lambda s, b: (0, 0)),
            pl.BlockSpec((1, 4 * H), lambda s, b: (0, 0)),
            pl.BlockSpec((4 * H, H), lambda s, b: (0, 0)),
            pl.BlockSpec((1, H), lambda s, b: (0, 0)),
            pl.BlockSpec((1, H), lambda s, b: (0, 0)),
            pl.BlockSpec((1, H), lambda s, b: (0, 0)),
        ],
        out_specs=pl.BlockSpec((TB, H), tok),
        out_shape=jax.ShapeDtypeStruct((N, H), jnp.float32),
        compiler_params=pltpu.CompilerParams(
            dimension_semantics=("arbitrary", "arbitrary")),
    )(tag2d, sub2d, x2d, pos, wi, bi, wo, bo, gamma, beta)


def kernel(inputs_embeds, path_tags_seq, path_subs_seq, tag_tables,
           subs_tables, W_inner, b_inner, W_out, b_out, pos_emb, ln_gamma,
           ln_beta):
    seq_len = inputs_embeds.shape[1]
    depths = jnp.arange(MAX_DEPTH, dtype=jnp.int32)
    tag_idx = (path_tags_seq + depths * TAG_V).reshape(R)
    sub_idx = (path_subs_seq + depths * SUB_V).reshape(R)
    tag_flat = tag_tables.reshape(MAX_DEPTH * TAG_V, PU)
    sub_flat = subs_tables.reshape(MAX_DEPTH * SUB_V, PU)

    x2d = inputs_embeds.reshape(N, H)
    pos = pos_emb[:seq_len]
    wi = W_inner.astype(jnp.bfloat16)
    bi = b_inner.reshape(1, 4 * H)
    wo = W_out.astype(jnp.bfloat16)
    bo = b_out.reshape(1, H)
    g = ln_gamma.reshape(1, H)
    bt = ln_beta.reshape(1, H)

    tag_cat, sub_cat = _sc_gather(tag_flat, sub_flat, tag_idx, sub_idx)
    out = _tc_fused(
        tag_cat.reshape(N, MAX_DEPTH * PU),
        sub_cat.reshape(N, MAX_DEPTH * PU),
        x2d, pos, wi, bi, wo, bo, g, bt)
    return out.reshape(B, S, H)


# speedup vs baseline: 1.3044x; 1.1100x over previous
import jax
import jax.numpy as jnp
from jax import lax
from jax.experimental import pallas as pl
from jax.experimental.pallas import tpu as pltpu
from jax.experimental.pallas import tpu_sc as plsc

B, S, H = 4, 2048, 1024
MAX_DEPTH = 16
PU = 64
TAG_V = 256
SUB_V = 1024
EPS = 1e-12

N = B * S
R = N * MAX_DEPTH

NC, NS = 2, 16
NW = NC * NS
CHUNK = 128


def _sc_gather(tag_tab_flat, sub_tab_flat, tag_idx_flat, sub_idx_flat):
    rows = tag_idx_flat.shape[0]
    dt = tag_tab_flat.dtype
    per_w = rows // NW
    n_chunks = per_w // CHUNK

    grp = 4
    n_grp = n_chunks // grp

    def body(tag_tab, sub_tab, tag_idx, sub_idx, tag_out, sub_out,
             idx_t, idx_s, rows_t, rows_s,
             sem_it, sem_is, sem_gt, sem_gs, sem_st, sem_ss):
        wid = lax.axis_index("s") * NC + lax.axis_index("c")
        base = wid * per_w

        def group(g, carry):
            goff = base + g * grp * CHUNK
            it, isx = [], []
            for b in range(grp):
                off = goff + b * CHUNK
                it.append(pltpu.async_copy(
                    tag_idx.at[pl.ds(off, CHUNK)], idx_t.at[b], sem_it.at[b]))
                isx.append(pltpu.async_copy(
                    sub_idx.at[pl.ds(off, CHUNK)], idx_s.at[b], sem_is.at[b]))
            gt, gs = [], []
            for b in range(grp):
                it[b].wait()
                gt.append(pltpu.async_copy(
                    tag_tab.at[idx_t.at[b]], rows_t.at[b], sem_gt.at[b]))
                isx[b].wait()
                gs.append(pltpu.async_copy(
                    sub_tab.at[idx_s.at[b]], rows_s.at[b], sem_gs.at[b]))
            st, ss = [], []
            for b in range(grp):
                off = goff + b * CHUNK
                gt[b].wait()
                st.append(pltpu.async_copy(
                    rows_t.at[b], tag_out.at[pl.ds(off, CHUNK)], sem_st.at[b]))
                gs[b].wait()
                ss.append(pltpu.async_copy(
                    rows_s.at[b], sub_out.at[pl.ds(off, CHUNK)], sem_ss.at[b]))
            for b in range(grp):
                st[b].wait()
                ss[b].wait()
            return carry

        lax.fori_loop(0, n_grp, group, 0)

    mesh = plsc.VectorSubcoreMesh(core_axis_name="c", subcore_axis_name="s")
    return pl.kernel(
        body,
        out_type=(jax.ShapeDtypeStruct((rows, PU), dt),
                  jax.ShapeDtypeStruct((rows, PU), dt)),
        mesh=mesh,
        scratch_types=[
            pltpu.VMEM((grp, CHUNK), jnp.int32),
            pltpu.VMEM((grp, CHUNK), jnp.int32),
            pltpu.VMEM((grp, CHUNK, PU), dt),
            pltpu.VMEM((grp, CHUNK, PU), dt),
            pltpu.SemaphoreType.DMA((grp,)),
            pltpu.SemaphoreType.DMA((grp,)),
            pltpu.SemaphoreType.DMA((grp,)),
            pltpu.SemaphoreType.DMA((grp,)),
            pltpu.SemaphoreType.DMA((grp,)),
            pltpu.SemaphoreType.DMA((grp,)),
        ],
        compiler_params=pltpu.CompilerParams(use_tc_tiling_on_sc=False),
    )(tag_tab_flat, sub_tab_flat, tag_idx_flat, sub_idx_flat)


TB = 512


def _tc_body(tag_ref, sub_ref, x_ref, pos_ref, wi_ref, bi_ref, wo_ref, bo_ref,
             g_ref, b_ref, o_ref):
    path = (tag_ref[...] + sub_ref[...]).astype(jnp.bfloat16)
    h1 = jnp.dot(path, wi_ref[...], preferred_element_type=jnp.float32)
    h1 = jnp.maximum(h1 + bi_ref[...], 0.0).astype(jnp.bfloat16)
    pe = jnp.dot(h1, wo_ref[...], preferred_element_type=jnp.float32)
    emb = x_ref[...] + pos_ref[...] + (pe + bo_ref[...])
    mean = jnp.mean(emb, axis=-1, keepdims=True)
    cent = emb - mean
    var = jnp.mean(cent * cent, axis=-1, keepdims=True)
    o_ref[...] = cent * lax.rsqrt(var + EPS) * g_ref[...] + b_ref[...]


def _tc_fused(tag2d, sub2d, x2d, pos, wi, bi, wo, bo, gamma, beta):
    grid = (S // TB, B)
    dp = MAX_DEPTH * PU
    tok = lambda s, b: (b * (S // TB) + s, 0)
    return pl.pallas_call(
        _tc_body,
        grid=grid,
        in_specs=[
            pl.BlockSpec((TB, dp), tok),
            pl.BlockSpec((TB, dp), tok),
            pl.BlockSpec((TB, H), tok),
            pl.BlockSpec((TB, H), lambda s, b: (s, 0)),
            pl.BlockSpec((dp, 4 * H), lambda s, b: (0, 0)),
            pl.BlockSpec((1, 4 * H), lambda s, b: (0, 0)),
            pl.BlockSpec((4 * H, H), lambda s, b: (0, 0)),
            pl.BlockSpec((1, H), lambda s, b: (0, 0)),
            pl.BlockSpec((1, H), lambda s, b: (0, 0)),
            pl.BlockSpec((1, H), lambda s, b: (0, 0)),
        ],
        out_specs=pl.BlockSpec((TB, H), tok),
        out_shape=jax.ShapeDtypeStruct((N, H), jnp.float32),
        compiler_params=pltpu.CompilerParams(
            dimension_semantics=("arbitrary", "arbitrary")),
    )(tag2d, sub2d, x2d, pos, wi, bi, wo, bo, gamma, beta)


def kernel(inputs_embeds, path_tags_seq, path_subs_seq, tag_tables,
           subs_tables, W_inner, b_inner, W_out, b_out, pos_emb, ln_gamma,
           ln_beta):
    seq_len = inputs_embeds.shape[1]
    depths = jnp.arange(MAX_DEPTH, dtype=jnp.int32)
    tag_idx = (path_tags_seq + depths * TAG_V).reshape(R)
    sub_idx = (path_subs_seq + depths * SUB_V).reshape(R)
    tag_flat = tag_tables.reshape(MAX_DEPTH * TAG_V, PU)
    sub_flat = subs_tables.reshape(MAX_DEPTH * SUB_V, PU)

    x2d = inputs_embeds.reshape(N, H)
    pos = pos_emb[:seq_len]
    wi = W_inner.astype(jnp.bfloat16)
    bi = b_inner.reshape(1, 4 * H)
    wo = W_out.astype(jnp.bfloat16)
    bo = b_out.reshape(1, H)
    g = ln_gamma.reshape(1, H)
    bt = ln_beta.reshape(1, H)

    tag_cat, sub_cat = _sc_gather(tag_flat, sub_flat, tag_idx, sub_idx)
    out = _tc_fused(
        tag_cat.reshape(N, MAX_DEPTH * PU),
        sub_cat.reshape(N, MAX_DEPTH * PU),
        x2d, pos, wi, bi, wo, bo, g, bt)
    return out.reshape(B, S, H)
